# initial kernel scaffold (unmeasured)
import jax
import jax.numpy as jnp
from jax import lax
from jax.experimental import pallas as pl
from jax.experimental.pallas import tpu as pltpu

T = 512
D = 1024
V_SHARD = 8192


def kernel(x, W, labels):
    labels2d = labels.reshape(T, 1)

    def body(x_ref, w_ref, lab_ref, out_ref, stats_ref, recv_ref, send_sem, recv_sem):
        my_x = lax.axis_index("x")
        my_y = lax.axis_index("y")
        my_z = lax.axis_index("z")
        partner = (1 - my_x, my_y, my_z)

        logits = jnp.dot(x_ref[:, :], w_ref[:, :], preferred_element_type=jnp.float32)

        m_loc = jnp.max(logits, axis=1)
        s_loc = jnp.sum(jnp.exp(logits - m_loc[:, None]), axis=1)
        col = lax.broadcasted_iota(jnp.int32, (T, V_SHARD), 1)
        rel = lab_ref[:, :] - my_x * V_SHARD
        l_loc = jnp.sum(jnp.where(col == rel, logits, 0.0), axis=1)

        stats_ref[0, :] = m_loc
        stats_ref[1, :] = s_loc
        stats_ref[2, :] = l_loc

        barrier_sem = pltpu.get_barrier_semaphore()
        pl.semaphore_signal(
            barrier_sem, inc=1, device_id=partner,
            device_id_type=pl.DeviceIdType.MESH,
        )
        pl.semaphore_wait(barrier_sem, 1)

        rdma = pltpu.make_async_remote_copy(
            src_ref=stats_ref,
            dst_ref=recv_ref,
            send_sem=send_sem,
            recv_sem=recv_sem,
            device_id=partner,
            device_id_type=pl.DeviceIdType.MESH,
        )
        rdma.start()
        rdma.wait()

        m_rem = recv_ref[0, :]
        s_rem = recv_ref[1, :]
        l_rem = recv_ref[2, :]
        m = jnp.maximum(m_loc, m_rem)
        s = s_loc * jnp.exp(m_loc - m) + s_rem * jnp.exp(m_rem - m)
        out_ref[:] = m + jnp.log(s) - (l_loc + l_rem)

    return pl.pallas_call(
        body,
        out_shape=jax.ShapeDtypeStruct((T,), jnp.float32),
        in_specs=[
            pl.BlockSpec(memory_space=pltpu.VMEM),
            pl.BlockSpec(memory_space=pltpu.VMEM),
            pl.BlockSpec(memory_space=pltpu.VMEM),
        ],
        out_specs=pl.BlockSpec(memory_space=pltpu.VMEM),
        scratch_shapes=[
            pltpu.VMEM((3, T), jnp.float32),
            pltpu.VMEM((3, T), jnp.float32),
            pltpu.SemaphoreType.DMA,
            pltpu.SemaphoreType.DMA,
        ],
        compiler_params=pltpu.CompilerParams(collective_id=0),
    )(x, W, labels2d)


# baseline (device time: 29765 ns/iter reference)
import jax
import jax.numpy as jnp
from jax import lax
from jax.experimental import pallas as pl
from jax.experimental.pallas import tpu as pltpu

T = 512
D = 1024
V_SHARD = 8192
VB = 1024
N_STEPS = V_SHARD // VB


def kernel(x, W, labels):
    labels2d = labels.reshape(T, 1)

    def body(x_ref, w_ref, lab_ref, out_ref, stats_ref, recv_ref, send_sem, recv_sem):
        j = pl.program_id(0)
        my_x = lax.axis_index("x")

        logits = jnp.dot(x_ref[:, :], w_ref[:, :], preferred_element_type=jnp.float32)

        m_chunk = jnp.max(logits, axis=1)
        col = lax.broadcasted_iota(jnp.int32, (T, VB), 1)
        rel = lab_ref[:, :] - my_x * V_SHARD - j * VB
        l_chunk = jnp.sum(jnp.where(col == rel, logits, 0.0), axis=1)

        @pl.when(j == 0)
        def _():
            s0 = jnp.sum(jnp.exp(logits - m_chunk[:, None]), axis=1)
            stats_ref[0, :] = m_chunk
            stats_ref[1, :] = s0
            stats_ref[2, :] = l_chunk

        @pl.when(j > 0)
        def _():
            m_old = stats_ref[0, :]
            m_new = jnp.maximum(m_old, m_chunk)
            s_chunk = jnp.sum(jnp.exp(logits - m_new[:, None]), axis=1)
            stats_ref[0, :] = m_new
            stats_ref[1, :] = stats_ref[1, :] * jnp.exp(m_old - m_new) + s_chunk
            stats_ref[2, :] = stats_ref[2, :] + l_chunk

        @pl.when(j == N_STEPS - 1)
        def _():
            my_y = lax.axis_index("y")
            my_z = lax.axis_index("z")
            partner = (1 - my_x, my_y, my_z)

            barrier_sem = pltpu.get_barrier_semaphore()
            pl.semaphore_signal(
                barrier_sem, inc=1, device_id=partner,
                device_id_type=pl.DeviceIdType.MESH,
            )
            pl.semaphore_wait(barrier_sem, 1)

            rdma = pltpu.make_async_remote_copy(
                src_ref=stats_ref,
                dst_ref=recv_ref,
                send_sem=send_sem,
                recv_sem=recv_sem,
                device_id=partner,
                device_id_type=pl.DeviceIdType.MESH,
            )
            rdma.start()
            rdma.wait()

            m_loc = stats_ref[0, :]
            s_loc = stats_ref[1, :]
            l_loc = stats_ref[2, :]
            m_rem = recv_ref[0, :]
            s_rem = recv_ref[1, :]
            l_rem = recv_ref[2, :]
            m = jnp.maximum(m_loc, m_rem)
            s = s_loc * jnp.exp(m_loc - m) + s_rem * jnp.exp(m_rem - m)
            out_ref[:] = m + jnp.log(s) - (l_loc + l_rem)

    return pl.pallas_call(
        body,
        grid=(N_STEPS,),
        out_shape=jax.ShapeDtypeStruct((T,), jnp.float32),
        in_specs=[
            pl.BlockSpec((T, D), lambda j: (0, 0), memory_space=pltpu.VMEM),
            pl.BlockSpec((D, VB), lambda j: (0, j), memory_space=pltpu.VMEM),
            pl.BlockSpec((T, 1), lambda j: (0, 0), memory_space=pltpu.VMEM),
        ],
        out_specs=pl.BlockSpec((T,), lambda j: (0,), memory_space=pltpu.VMEM),
        scratch_shapes=[
            pltpu.VMEM((3, T), jnp.float32),
            pltpu.VMEM((3, T), jnp.float32),
            pltpu.SemaphoreType.DMA,
            pltpu.SemaphoreType.DMA,
        ],
        compiler_params=pltpu.CompilerParams(
            collective_id=0,
            dimension_semantics=("arbitrary",),
        ),
    )(x, W, labels2d)
